# Initial kernel scaffold; baseline (speedup 1.0000x reference)
import jax
import jax.numpy as jnp
from jax.experimental import pallas as pl


def kernel(x, edge_index, batch, Wrel0, brel0, Wroot0, Wrel1, brel1, Wroot1, Wrel2, brel2, Wroot2, Wlin1, blin1, Wlin2, blin2):
    raise NotImplementedError("write your pallas kernel here")


# speedup vs baseline: 5.7194x; 5.7194x over previous
import functools

import jax
import jax.numpy as jnp
from jax import lax
from jax.experimental import pallas as pl
from jax.experimental.pallas import tpu as pltpu
from jax.experimental.pallas import tpu_sc as plsc

N = 10000
E = 320000
D = 128
G = 64
OUT = 10

NC = 2
NS = 16
CH = 128

E_PER_SC = E // NC
NCHUNK_SC = E_PER_SC // CH
RPT = 624
RPT_LAST = N - RPT * (NS - 1)
ZROWS = 16


def _segment_sum_sc(x, src, dst):
    mesh = plsc.VectorSubcoreMesh(core_axis_name="c", subcore_axis_name="s")

    @functools.partial(
        pl.kernel,
        mesh=mesh,
        out_type=jax.ShapeDtypeStruct((NC, N, D), jnp.float32),
        scratch_types=[
            pltpu.VMEM((CH,), jnp.int32),
            pltpu.VMEM((CH,), jnp.int32),
            pltpu.VMEM((CH, D), jnp.float32),
            pltpu.VMEM((ZROWS, D), jnp.float32),
            pltpu.VMEM_SHARED((N, D), jnp.float32),
            pltpu.SemaphoreType.DMA,
        ],
    )
    def k(x_hbm, src_hbm, dst_hbm, out_hbm, src_v, dst_v, rows_v, zb, acc, sem):
        cid = lax.axis_index("c")
        sid = lax.axis_index("s")

        def zrow(r, carry):
            for c8 in range(D // 16):
                zb[r, pl.ds(c8 * 16, 16)] = jnp.zeros((16,), jnp.float32)
            return carry

        lax.fori_loop(0, ZROWS, zrow, 0)
        rbase = sid * RPT
        nz = jnp.where(sid == NS - 1, RPT_LAST // ZROWS, RPT // ZROWS)

        def zcp(i, carry):
            pltpu.sync_copy(zb, acc.at[pl.ds(rbase + i * ZROWS, ZROWS)])
            return carry

        lax.fori_loop(0, nz, zcp, 0)
        plsc.subcore_barrier()

        nch = jnp.where(sid < 2, 79, 78)
        base_e = cid * E_PER_SC + (sid * 78 + jnp.minimum(sid, 2)) * CH

        def body(j, carry):
            b = base_e + j * CH
            pltpu.sync_copy(src_hbm.at[pl.ds(b, CH)], src_v)
            pltpu.sync_copy(dst_hbm.at[pl.ds(b, CH)], dst_v)
            pltpu.async_copy(x_hbm.at[src_v], rows_v, sem).wait()
            pltpu.sync_copy(rows_v, acc.at[dst_v], add=True)
            return carry

        lax.fori_loop(0, nch, body, 0)
        plsc.subcore_barrier()

        @pl.when(sid < NS - 1)
        def _():
            pltpu.sync_copy(
                acc.at[pl.ds(rbase, RPT)],
                out_hbm.at[cid, pl.ds(rbase, RPT)],
            )

        @pl.when(sid == NS - 1)
        def _():
            pltpu.sync_copy(
                acc.at[pl.ds(rbase, RPT_LAST)],
                out_hbm.at[cid, pl.ds(rbase, RPT_LAST)],
            )

    return k(x, src, dst)


def _gconv_dense_tc(partials, x, Wrel, Wroot, brel):
    RB = 1000

    def body(p_ref, x_ref, wr_ref, wo_ref, b_ref, o_ref):
        agg = p_ref[0] + p_ref[1]
        h = jnp.dot(agg, wr_ref[...], preferred_element_type=jnp.float32)
        h = h + jnp.dot(x_ref[...], wo_ref[...], preferred_element_type=jnp.float32)
        o_ref[...] = jnp.maximum(h + b_ref[...], 0.0)

    return pl.pallas_call(
        body,
        grid=(N // RB,),
        in_specs=[
            pl.BlockSpec((NC, RB, D), lambda i: (0, i, 0)),
            pl.BlockSpec((RB, D), lambda i: (i, 0)),
            pl.BlockSpec((D, D), lambda i: (0, 0)),
            pl.BlockSpec((D, D), lambda i: (0, 0)),
            pl.BlockSpec((1, D), lambda i: (0, 0)),
        ],
        out_specs=pl.BlockSpec((RB, D), lambda i: (i, 0)),
        out_shape=jax.ShapeDtypeStruct((N, D), jnp.float32),
    )(partials, x, Wrel, Wroot, brel.reshape(1, D))


def _classifier_pool_tc(h, Wlin1, blin1, batch3, Wfin):
    RB = 1000
    GRID = N // RB

    def body(h_ref, b3_ref, w1_ref, b1_ref, wf_ref, o_ref, acc_ref):
        i = pl.program_id(0)

        @pl.when(i == 0)
        def _():
            acc_ref[...] = jnp.zeros_like(acc_ref)

        z = jnp.dot(h_ref[...], w1_ref[...], preferred_element_type=jnp.float32)
        z = jnp.maximum(z + b1_ref[...], 0.0)
        segs = lax.broadcasted_iota(jnp.int32, (G, RB), 0)
        oh = (segs == b3_ref[0]).astype(jnp.float32)
        acc_ref[:, :D] += jnp.dot(oh, z, preferred_element_type=jnp.float32)
        cnt = jnp.sum(oh, axis=1)
        col = lax.broadcasted_iota(jnp.int32, (G, 16), 1)
        acc_ref[:, D:] += jnp.where(col == 0, cnt[:, None], 0.0)

        @pl.when(i == GRID - 1)
        def _():
            o_ref[...] = jnp.dot(
                acc_ref[...], wf_ref[...], preferred_element_type=jnp.float32
            )

    return pl.pallas_call(
        body,
        grid=(GRID,),
        in_specs=[
            pl.BlockSpec((RB, D), lambda i: (i, 0)),
            pl.BlockSpec((1, 1, RB), lambda i: (i, 0, 0)),
            pl.BlockSpec((D, D), lambda i: (0, 0)),
            pl.BlockSpec((1, D), lambda i: (0, 0)),
            pl.BlockSpec((D + 16, D), lambda i: (0, 0)),
        ],
        out_specs=pl.BlockSpec((G, D), lambda i: (0, 0)),
        out_shape=jax.ShapeDtypeStruct((G, D), jnp.float32),
        scratch_shapes=[pltpu.VMEM((G, D + 16), jnp.float32)],
    )(h, batch3, Wlin1, blin1.reshape(1, D), Wfin)


def kernel(x, edge_index, batch,
           Wrel0, brel0, Wroot0,
           Wrel1, brel1, Wroot1,
           Wrel2, brel2, Wroot2,
           Wlin1, blin1, Wlin2, blin2):
    src = edge_index[0]
    dst = edge_index[1]

    h = x
    for Wrel, brel, Wroot in (
        (Wrel0, brel0, Wroot0),
        (Wrel1, brel1, Wroot1),
        (Wrel2, brel2, Wroot2),
    ):
        partials = _segment_sum_sc(h, src, dst)
        h = _gconv_dense_tc(partials, h, Wrel, Wroot, brel)

    Wfin = jnp.zeros((D + 16, D), jnp.float32)
    Wfin = Wfin.at[:D, :OUT].set(Wlin2)
    Wfin = Wfin.at[D, :OUT].set(blin2)
    batch3 = batch.reshape(N // 1000, 1, 1000)

    out = _classifier_pool_tc(h, Wlin1, blin1, batch3, Wfin)
    return out[:, :OUT]


# speedup vs baseline: 10.2353x; 1.7896x over previous
import functools

import jax
import jax.numpy as jnp
from jax import lax
from jax.experimental import pallas as pl
from jax.experimental.pallas import tpu as pltpu
from jax.experimental.pallas import tpu_sc as plsc

N = 10000
E = 320000
D = 128
G = 64
OUT = 10

NC = 2
NS = 16
CH = 128

E_PER_SC = E // NC
NCHUNK_SC = E_PER_SC // CH
RPT = 624
RPT_LAST = N - RPT * (NS - 1)
ZROWS = 16


NCH_MAIN = 78


def _segment_sum_sc(x, src, dst):
    mesh = plsc.VectorSubcoreMesh(core_axis_name="c", subcore_axis_name="s")

    @functools.partial(
        pl.kernel,
        mesh=mesh,
        out_type=jax.ShapeDtypeStruct((NC, N, D), jnp.float32),
        scratch_types=[
            pltpu.VMEM((79 * CH,), jnp.int32),
            pltpu.VMEM((CH,), jnp.int32),
            pltpu.VMEM((CH,), jnp.int32),
            pltpu.VMEM((CH, D), jnp.float32),
            pltpu.VMEM((CH, D), jnp.float32),
            pltpu.VMEM((ZROWS, D), jnp.float32),
            pltpu.VMEM_SHARED((N, D), jnp.float32),
            pltpu.SemaphoreType.DMA,
            pltpu.SemaphoreType.DMA,
            pltpu.SemaphoreType.DMA,
            pltpu.SemaphoreType.DMA,
            pltpu.SemaphoreType.DMA,
        ],
    )
    def k(x_hbm, src_hbm, dst_hbm, out_hbm, src_all, dv0, dv1, buf0, buf1,
          zb, acc, sg0, sg1, ss0, ss1, sz):
        cid = lax.axis_index("c")
        sid = lax.axis_index("s")
        dv = (dv0, dv1)
        buf = (buf0, buf1)
        sg = (sg0, sg1)
        ss = (ss0, ss1)

        nch = jnp.where(sid < 2, 79, NCH_MAIN)
        base_e = cid * E_PER_SC + (sid * NCH_MAIN + jnp.minimum(sid, 2)) * CH

        def start_chunk(i, slot):
            pltpu.async_copy(dst_hbm.at[pl.ds(base_e + i * CH, CH)],
                             dv[slot], sg[slot])
            pltpu.async_copy(x_hbm.at[src_all.at[pl.ds(i * CH, CH)]],
                             buf[slot], sg[slot])

        def wait_chunk(slot):
            pltpu.make_async_copy(dst_hbm.at[pl.ds(0, CH)], dv[slot],
                                  sg[slot]).wait()
            pltpu.make_async_copy(x_hbm.at[pl.ds(0, CH)], buf[slot],
                                  sg[slot]).wait()

        def wait_scatter(slot):
            pltpu.make_async_copy(buf[slot], acc.at[dv[slot]],
                                  ss[slot]).wait()

        pltpu.sync_copy(src_hbm.at[pl.ds(base_e, NCH_MAIN * CH)],
                        src_all.at[pl.ds(0, NCH_MAIN * CH)])

        @pl.when(sid < 2)
        def _():
            pltpu.sync_copy(src_hbm.at[pl.ds(base_e + NCH_MAIN * CH, CH)],
                            src_all.at[pl.ds(NCH_MAIN * CH, CH)])

        start_chunk(0, 0)

        def zrow(r, carry):
            for c8 in range(D // 16):
                zb[r, pl.ds(c8 * 16, 16)] = jnp.zeros((16,), jnp.float32)
            return carry

        lax.fori_loop(0, ZROWS, zrow, 0)
        rbase = sid * RPT
        nz = jnp.where(sid == NS - 1, RPT_LAST // ZROWS, RPT // ZROWS)

        def zcp(i, carry):
            pltpu.async_copy(zb, acc.at[pl.ds(rbase + i * ZROWS, ZROWS)], sz)
            return carry

        lax.fori_loop(0, nz, zcp, 0)

        def zdrain(i, carry):
            pltpu.make_async_copy(zb, acc.at[pl.ds(rbase, ZROWS)], sz).wait()
            return carry

        lax.fori_loop(0, nz, zdrain, 0)
        plsc.subcore_barrier()

        def pair(g, carry):
            for half in (0, 1):
                i = g * 2 + half
                wait_chunk(half)

                @pl.when(i >= 1)
                def _():
                    wait_scatter(1 - half)

                @pl.when(i + 1 < NCH_MAIN)
                def _():
                    start_chunk(i + 1, 1 - half)

                pltpu.async_copy(buf[half], acc.at[dv[half]], ss[half],
                                 add=True)
            return carry

        lax.fori_loop(0, NCH_MAIN // 2, pair, 0)
        wait_scatter(1)

        @pl.when(nch > NCH_MAIN)
        def _():
            start_chunk(NCH_MAIN, 0)
            wait_chunk(0)
            pltpu.async_copy(buf[0], acc.at[dv[0]], ss[0], add=True)
            wait_scatter(0)

        plsc.subcore_barrier()

        @pl.when(sid < NS - 1)
        def _():
            pltpu.sync_copy(
                acc.at[pl.ds(rbase, RPT)],
                out_hbm.at[cid, pl.ds(rbase, RPT)],
            )

        @pl.when(sid == NS - 1)
        def _():
            pltpu.sync_copy(
                acc.at[pl.ds(rbase, RPT_LAST)],
                out_hbm.at[cid, pl.ds(rbase, RPT_LAST)],
            )

    return k(x, src, dst)


def _gconv_dense_tc(partials, x, Wrel, Wroot, brel):
    RB = 1000

    def body(p_ref, x_ref, wr_ref, wo_ref, b_ref, o_ref):
        agg = p_ref[0] + p_ref[1]
        h = jnp.dot(agg, wr_ref[...], preferred_element_type=jnp.float32)
        h = h + jnp.dot(x_ref[...], wo_ref[...], preferred_element_type=jnp.float32)
        o_ref[...] = jnp.maximum(h + b_ref[...], 0.0)

    return pl.pallas_call(
        body,
        grid=(N // RB,),
        in_specs=[
            pl.BlockSpec((NC, RB, D), lambda i: (0, i, 0)),
            pl.BlockSpec((RB, D), lambda i: (i, 0)),
            pl.BlockSpec((D, D), lambda i: (0, 0)),
            pl.BlockSpec((D, D), lambda i: (0, 0)),
            pl.BlockSpec((1, D), lambda i: (0, 0)),
        ],
        out_specs=pl.BlockSpec((RB, D), lambda i: (i, 0)),
        out_shape=jax.ShapeDtypeStruct((N, D), jnp.float32),
    )(partials, x, Wrel, Wroot, brel.reshape(1, D))


def _classifier_pool_tc(h, Wlin1, blin1, batch3, Wfin):
    RB = 1000
    GRID = N // RB

    def body(h_ref, b3_ref, w1_ref, b1_ref, wf_ref, o_ref, acc_ref):
        i = pl.program_id(0)

        @pl.when(i == 0)
        def _():
            acc_ref[...] = jnp.zeros_like(acc_ref)

        z = jnp.dot(h_ref[...], w1_ref[...], preferred_element_type=jnp.float32)
        z = jnp.maximum(z + b1_ref[...], 0.0)
        segs = lax.broadcasted_iota(jnp.int32, (G, RB), 0)
        oh = (segs == b3_ref[0]).astype(jnp.float32)
        acc_ref[:, :D] += jnp.dot(oh, z, preferred_element_type=jnp.float32)
        cnt = jnp.sum(oh, axis=1)
        col = lax.broadcasted_iota(jnp.int32, (G, 16), 1)
        acc_ref[:, D:] += jnp.where(col == 0, cnt[:, None], 0.0)

        @pl.when(i == GRID - 1)
        def _():
            o_ref[...] = jnp.dot(
                acc_ref[...], wf_ref[...], preferred_element_type=jnp.float32
            )

    return pl.pallas_call(
        body,
        grid=(GRID,),
        in_specs=[
            pl.BlockSpec((RB, D), lambda i: (i, 0)),
            pl.BlockSpec((1, 1, RB), lambda i: (i, 0, 0)),
            pl.BlockSpec((D, D), lambda i: (0, 0)),
            pl.BlockSpec((1, D), lambda i: (0, 0)),
            pl.BlockSpec((D + 16, D), lambda i: (0, 0)),
        ],
        out_specs=pl.BlockSpec((G, D), lambda i: (0, 0)),
        out_shape=jax.ShapeDtypeStruct((G, D), jnp.float32),
        scratch_shapes=[pltpu.VMEM((G, D + 16), jnp.float32)],
    )(h, batch3, Wlin1, blin1.reshape(1, D), Wfin)


def kernel(x, edge_index, batch,
           Wrel0, brel0, Wroot0,
           Wrel1, brel1, Wroot1,
           Wrel2, brel2, Wroot2,
           Wlin1, blin1, Wlin2, blin2):
    src = edge_index[0]
    dst = edge_index[1]

    h = x
    for Wrel, brel, Wroot in (
        (Wrel0, brel0, Wroot0),
        (Wrel1, brel1, Wroot1),
        (Wrel2, brel2, Wroot2),
    ):
        partials = _segment_sum_sc(h, src, dst)
        h = _gconv_dense_tc(partials, h, Wrel, Wroot, brel)

    Wfin = jnp.zeros((D + 16, D), jnp.float32)
    Wfin = Wfin.at[:D, :OUT].set(Wlin2)
    Wfin = Wfin.at[D, :OUT].set(blin2)
    batch3 = batch.reshape(N // 1000, 1, 1000)

    out = _classifier_pool_tc(h, Wlin1, blin1, batch3, Wfin)
    return out[:, :OUT]
